# baseline (speedup 1.0000x reference)
import numpy as np
import jax
import jax.numpy as jnp
from jax.experimental import pallas as pl
from jax.experimental.pallas import tpu as pltpu

LANE = 128
BIMG = 512
H1, W1C, K = 32, 32, 5
C0, C1, C2 = 3, 6, 16
HO1 = H1 - K + 1
HP1 = HO1 // 2
HO2 = HP1 - K + 1
HP2 = HO2 // 2


def _np_consts():
    e1a = np.zeros((K, C0, C0 * W1C, HO1), np.float32)
    for dw in range(K):
        for ci in range(C0):
            for c in range(HO1):
                e1a[dw, ci, W1C * ci + c + dw, c] = 1.0
    e2a = np.zeros((HO1, C1, 2 * LANE), np.float32)
    for c in range(HO1):
        for co in range(C1):
            e2a[c, co, C1 * c + co] = 1.0
    e1b = np.zeros((K, C1, LANE, HO2), np.float32)
    for dw in range(K):
        for ci in range(C1):
            for c in range(HO2):
                e1b[dw, ci, C1 * (c + dw) + ci, c] = 1.0
    e2b = np.zeros((HO2, C2, 2 * LANE), np.float32)
    for c in range(HO2):
        for co in range(C2):
            e2b[c, co, C2 * c + co] = 1.0
    s1e = np.zeros((2 * LANE, LANE), np.float32)
    s1o = np.zeros((2 * LANE, LANE), np.float32)
    for c2 in range(HP1):
        for k in range(C1):
            s1e[C1 * (2 * c2) + k, C1 * c2 + k] = 1.0
            s1o[C1 * (2 * c2 + 1) + k, C1 * c2 + k] = 1.0
    s2e = np.zeros((2 * LANE, LANE), np.float32)
    s2o = np.zeros((2 * LANE, LANE), np.float32)
    for c2 in range(HP2):
        for k in range(C2):
            s2e[C2 * (2 * c2) + k, C2 * c2 + k] = 1.0
            s2o[C2 * (2 * c2 + 1) + k, C2 * c2 + k] = 1.0
    mb1 = np.zeros((LANE, 2 * LANE), np.float32)
    for c in range(HO1):
        for k in range(C1):
            mb1[k, C1 * c + k] = 1.0
    mb2 = np.zeros((LANE, 2 * LANE), np.float32)
    for c in range(HO2):
        for k in range(C2):
            mb2[k, C2 * c + k] = 1.0
    return e1a, e2a, e1b, e2b, s1e, s1o, s2e, s2o, mb1, mb2


_E1A, _E2A, _E1B, _E2B, _S1E, _S1O, _S2E, _S2O, _MB1, _MB2 = _np_consts()


def _lenet_kernel(x_ref, wa_ref, wa4_ref, ba_ref, s1e_ref, s1o_ref,
                  wb_ref, wb4_ref, bb_ref, s2e_ref, s2o_ref,
                  wf1_ref, wf14_ref, bf1_ref, wf2_ref, bf2_ref,
                  wf3_ref, bf3_ref, o_ref):
    b = x_ref.shape[1]
    xf = x_ref[...].reshape(H1 * b, LANE)

    bf16 = jnp.bfloat16
    x01 = jnp.concatenate([xf[0:HO1 * b], xf[b:(1 + HO1) * b]], axis=1)
    x23 = jnp.concatenate([xf[2 * b:(2 + HO1) * b],
                           xf[3 * b:(3 + HO1) * b]], axis=1)
    acc = (jnp.dot(x01, wa_ref[0], preferred_element_type=jnp.float32)
           + jnp.dot(x23, wa_ref[1], preferred_element_type=jnp.float32)
           + jnp.dot(xf[4 * b:(4 + HO1) * b], wa4_ref[...],
                     preferred_element_type=jnp.float32))
    acc = jnp.maximum(acc + ba_ref[...], 0.0)

    a3 = acc.reshape(HP1, 2, b, 2 * LANE)
    mrow = jnp.maximum(a3[:, 0, :, :], a3[:, 1, :, :])
    mrowb = mrow.reshape(HP1 * b, 2 * LANE).astype(bf16)
    p1 = jnp.maximum(
        jnp.dot(mrowb, s1e_ref[...], preferred_element_type=jnp.float32),
        jnp.dot(mrowb, s1o_ref[...], preferred_element_type=jnp.float32))
    p1b = p1.astype(bf16)

    p01 = jnp.concatenate([p1b[0:HO2 * b], p1b[b:(1 + HO2) * b]], axis=1)
    p23 = jnp.concatenate([p1b[2 * b:(2 + HO2) * b],
                           p1b[3 * b:(3 + HO2) * b]], axis=1)
    acc2 = (jnp.dot(p01, wb_ref[0], preferred_element_type=jnp.float32)
            + jnp.dot(p23, wb_ref[1], preferred_element_type=jnp.float32)
            + jnp.dot(p1b[4 * b:(4 + HO2) * b], wb4_ref[...],
                      preferred_element_type=jnp.float32))
    acc2 = jnp.maximum(acc2 + bb_ref[...], 0.0)

    a32 = acc2.reshape(HP2, 2, b, 2 * LANE)
    mrow2 = jnp.maximum(a32[:, 0, :, :], a32[:, 1, :, :])
    mrow2b = mrow2.reshape(HP2 * b, 2 * LANE).astype(bf16)
    p2 = jnp.maximum(
        jnp.dot(mrow2b, s2e_ref[...], preferred_element_type=jnp.float32),
        jnp.dot(mrow2b, s2o_ref[...], preferred_element_type=jnp.float32))
    p2b = p2.astype(bf16)

    f01 = jnp.concatenate([p2b[0:b], p2b[b:2 * b]], axis=1)
    f23 = jnp.concatenate([p2b[2 * b:3 * b], p2b[3 * b:4 * b]], axis=1)
    h = (jnp.dot(f01, wf1_ref[0], preferred_element_type=jnp.float32)
         + jnp.dot(f23, wf1_ref[1], preferred_element_type=jnp.float32)
         + jnp.dot(p2b[4 * b:5 * b], wf14_ref[...],
                   preferred_element_type=jnp.float32))
    h = jnp.maximum(h + bf1_ref[...], 0.0).astype(bf16)
    h = jnp.dot(h, wf2_ref[...], preferred_element_type=jnp.float32)
    h = jnp.maximum(h + bf2_ref[...], 0.0).astype(bf16)
    y = jnp.dot(h, wf3_ref[...], preferred_element_type=jnp.float32)
    o_ref[...] = (y + bf3_ref[...])[:, 0:10]


@jax.jit
def kernel(x, c1_w, c1_b, c1_p, c2_w, c2_b, c2_p,
           w1, b1, w2, b2, w3, b3):
    del c1_p, c2_p
    n = x.shape[0]

    bf16 = jnp.bfloat16
    wt1 = c1_w.reshape(K, K, LANE, LANE)[:, :, :C0, :C1]
    wa = jnp.einsum('wilc,hwio,com->hlm', _E1A, wt1, _E2A)
    wa = jnp.pad(wa, ((0, 0), (0, LANE - C0 * W1C), (0, 0))).astype(bf16)
    wt2 = c2_w.reshape(K, K, LANE, LANE)[:, :, :C1, :C2]
    wb = jnp.einsum('wilc,hwio,com->hlm', _E1B, wt2, _E2B).astype(bf16)
    ba = c1_b @ _MB1
    bb = c2_b @ _MB2
    wf1 = w1.reshape(K, K, LANE, LANE)[:, :, :C2, :].reshape(K, K * C2, LANE)
    wf1 = jnp.pad(wf1, ((0, 0), (0, LANE - K * C2), (0, 0))).astype(bf16)

    def _pairs(w5):
        return jnp.stack([jnp.concatenate([w5[0], w5[1]], axis=0),
                          jnp.concatenate([w5[2], w5[3]], axis=0)])
    wap, wa4 = _pairs(wa), wa[4]
    wbp, wb4 = _pairs(wb), wb[4]
    wf1p, wf14 = _pairs(wf1), wf1[4]

    xv = jnp.pad(
        jnp.transpose(x.astype(bf16), (2, 0, 1, 3)).reshape(H1, n, C0 * W1C),
        ((0, 0), (0, 0), (0, LANE - C0 * W1C)))

    out = pl.pallas_call(
        _lenet_kernel,
        out_shape=jax.ShapeDtypeStruct((n, 10), jnp.float32),
        grid=(n // BIMG,),
        in_specs=[
            pl.BlockSpec((H1, BIMG, LANE), lambda i: (0, i, 0)),
            pl.BlockSpec((2, 2 * LANE, 2 * LANE), lambda i: (0, 0, 0)),
            pl.BlockSpec((LANE, 2 * LANE), lambda i: (0, 0)),
            pl.BlockSpec((1, 2 * LANE), lambda i: (0, 0)),
            pl.BlockSpec((2 * LANE, LANE), lambda i: (0, 0)),
            pl.BlockSpec((2 * LANE, LANE), lambda i: (0, 0)),
            pl.BlockSpec((2, 2 * LANE, 2 * LANE), lambda i: (0, 0, 0)),
            pl.BlockSpec((LANE, 2 * LANE), lambda i: (0, 0)),
            pl.BlockSpec((1, 2 * LANE), lambda i: (0, 0)),
            pl.BlockSpec((2 * LANE, LANE), lambda i: (0, 0)),
            pl.BlockSpec((2 * LANE, LANE), lambda i: (0, 0)),
            pl.BlockSpec((2, 2 * LANE, LANE), lambda i: (0, 0, 0)),
            pl.BlockSpec((LANE, LANE), lambda i: (0, 0)),
            pl.BlockSpec((1, LANE), lambda i: (0, 0)),
            pl.BlockSpec((LANE, LANE), lambda i: (0, 0)),
            pl.BlockSpec((1, LANE), lambda i: (0, 0)),
            pl.BlockSpec((LANE, LANE), lambda i: (0, 0)),
            pl.BlockSpec((1, LANE), lambda i: (0, 0)),
        ],
        out_specs=pl.BlockSpec((BIMG, 10), lambda i: (i, 0)),
        compiler_params=pltpu.CompilerParams(
            dimension_semantics=("parallel",),
            allow_input_fusion=(True,) * 18,
            vmem_limit_bytes=60 * 1024 * 1024,
        ),
    )(xv, wap, wa4, ba,
      jnp.asarray(_S1E, bf16), jnp.asarray(_S1O, bf16),
      wbp, wb4, bb,
      jnp.asarray(_S2E, bf16), jnp.asarray(_S2O, bf16),
      wf1p, wf14, b1, w2.astype(bf16), b2, w3.astype(bf16), b3)
    return out


# speedup vs baseline: 1.0010x; 1.0010x over previous
import numpy as np
import jax
import jax.numpy as jnp
from jax.experimental import pallas as pl
from jax.experimental.pallas import tpu as pltpu

LANE = 128
BIMG = 512
H1, W1C, K = 32, 32, 5
C0, C1, C2 = 3, 6, 16
HO1 = H1 - K + 1
HP1 = HO1 // 2
HO2 = HP1 - K + 1
HP2 = HO2 // 2


def _np_consts():
    e1a = np.zeros((K, C0, C0 * W1C, HO1), np.float32)
    for dw in range(K):
        for ci in range(C0):
            for c in range(HO1):
                e1a[dw, ci, W1C * ci + c + dw, c] = 1.0
    e2a = np.zeros((HO1, C1, 2 * LANE), np.float32)
    for c in range(HO1):
        for co in range(C1):
            e2a[c, co, C1 * c + co] = 1.0
    e1b = np.zeros((K, C1, LANE, HO2), np.float32)
    for dw in range(K):
        for ci in range(C1):
            for c in range(HO2):
                e1b[dw, ci, C1 * (c + dw) + ci, c] = 1.0
    e2b = np.zeros((HO2, C2, 2 * LANE), np.float32)
    for c in range(HO2):
        for co in range(C2):
            e2b[c, co, C2 * c + co] = 1.0
    s1e = np.zeros((2 * LANE, LANE), np.float32)
    s1o = np.zeros((2 * LANE, LANE), np.float32)
    for c2 in range(HP1):
        for k in range(C1):
            s1e[C1 * (2 * c2) + k, C1 * c2 + k] = 1.0
            s1o[C1 * (2 * c2 + 1) + k, C1 * c2 + k] = 1.0
    s2e = np.zeros((2 * LANE, LANE), np.float32)
    s2o = np.zeros((2 * LANE, LANE), np.float32)
    for c2 in range(HP2):
        for k in range(C2):
            s2e[C2 * (2 * c2) + k, C2 * c2 + k] = 1.0
            s2o[C2 * (2 * c2 + 1) + k, C2 * c2 + k] = 1.0
    mb1 = np.zeros((LANE, 2 * LANE), np.float32)
    for c in range(HO1):
        for k in range(C1):
            mb1[k, C1 * c + k] = 1.0
    mb2 = np.zeros((LANE, 2 * LANE), np.float32)
    for c in range(HO2):
        for k in range(C2):
            mb2[k, C2 * c + k] = 1.0
    return e1a, e2a, e1b, e2b, s1e, s1o, s2e, s2o, mb1, mb2


_E1A, _E2A, _E1B, _E2B, _S1E, _S1O, _S2E, _S2O, _MB1, _MB2 = _np_consts()


def _lenet_kernel(x_ref, wa_ref, wa4_ref, ba_ref, s1e_ref, s1o_ref,
                  wb_ref, wb4_ref, bb_ref, s2e_ref, s2o_ref,
                  wf1_ref, wf14_ref, bf1_ref, wf2_ref, bf2_ref,
                  wf3_ref, bf3_ref, o_ref):
    b = x_ref.shape[1]
    xf = x_ref[...].reshape(H1 * b, LANE)

    bf16 = jnp.bfloat16
    x01 = jnp.concatenate([xf[0:HO1 * b], xf[b:(1 + HO1) * b]], axis=1)
    x23 = jnp.concatenate([xf[2 * b:(2 + HO1) * b],
                           xf[3 * b:(3 + HO1) * b]], axis=1)
    acc = (jnp.dot(x01, wa_ref[0], preferred_element_type=jnp.float32)
           + jnp.dot(x23, wa_ref[1], preferred_element_type=jnp.float32)
           + jnp.dot(xf[4 * b:(4 + HO1) * b], wa4_ref[...],
                     preferred_element_type=jnp.float32))
    acc = jnp.maximum(acc + ba_ref[...], 0.0)

    a3 = acc.reshape(HP1, 2, b, 2 * LANE)
    mrow = jnp.maximum(a3[:, 0, :, :], a3[:, 1, :, :])
    mrowb = mrow.reshape(HP1 * b, 2 * LANE).astype(bf16)
    p1 = jnp.maximum(
        jnp.dot(mrowb, s1e_ref[...], preferred_element_type=jnp.float32),
        jnp.dot(mrowb, s1o_ref[...], preferred_element_type=jnp.float32))
    p1b = p1.astype(bf16)

    p01 = jnp.concatenate([p1b[0:HO2 * b], p1b[b:(1 + HO2) * b]], axis=1)
    p23 = jnp.concatenate([p1b[2 * b:(2 + HO2) * b],
                           p1b[3 * b:(3 + HO2) * b]], axis=1)
    acc2 = (jnp.dot(p01, wb_ref[0], preferred_element_type=jnp.float32)
            + jnp.dot(p23, wb_ref[1], preferred_element_type=jnp.float32)
            + jnp.dot(p1b[4 * b:(4 + HO2) * b], wb4_ref[...],
                      preferred_element_type=jnp.float32))
    acc2 = jnp.maximum(acc2 + bb_ref[...], 0.0)

    a32 = acc2.reshape(HP2, 2, b, 2 * LANE)
    mrow2 = jnp.maximum(a32[:, 0, :, :], a32[:, 1, :, :])
    mrow2b = mrow2.reshape(HP2 * b, 2 * LANE).astype(bf16)
    p2 = jnp.maximum(
        jnp.dot(mrow2b, s2e_ref[...], preferred_element_type=jnp.float32),
        jnp.dot(mrow2b, s2o_ref[...], preferred_element_type=jnp.float32))
    p2b = p2.astype(bf16)

    f01 = jnp.concatenate([p2b[0:b], p2b[b:2 * b]], axis=1)
    f23 = jnp.concatenate([p2b[2 * b:3 * b], p2b[3 * b:4 * b]], axis=1)
    h = (jnp.dot(f01, wf1_ref[0], preferred_element_type=jnp.float32)
         + jnp.dot(f23, wf1_ref[1], preferred_element_type=jnp.float32)
         + jnp.dot(p2b[4 * b:5 * b], wf14_ref[...],
                   preferred_element_type=jnp.float32))
    h = jnp.maximum(h + bf1_ref[...], 0.0).astype(bf16)
    h = jnp.dot(h, wf2_ref[...], preferred_element_type=jnp.float32)
    h = jnp.maximum(h + bf2_ref[...], 0.0).astype(bf16)
    y = jnp.dot(h, wf3_ref[...], preferred_element_type=jnp.float32)
    o_ref[...] = (y + bf3_ref[...])[:, 0:10]


@jax.jit
def kernel(x, c1_w, c1_b, c1_p, c2_w, c2_b, c2_p,
           w1, b1, w2, b2, w3, b3):
    del c1_p, c2_p
    n = x.shape[0]

    bf16 = jnp.bfloat16
    wt1 = c1_w.reshape(K, K, LANE, LANE)[:, :, :C0, :C1]
    wa = jnp.einsum('wilc,hwio,com->hlm', _E1A, wt1, _E2A)
    wa = jnp.pad(wa, ((0, 0), (0, LANE - C0 * W1C), (0, 0))).astype(bf16)
    wt2 = c2_w.reshape(K, K, LANE, LANE)[:, :, :C1, :C2]
    wb = jnp.einsum('wilc,hwio,com->hlm', _E1B, wt2, _E2B).astype(bf16)
    ba = c1_b @ _MB1
    bb = c2_b @ _MB2
    wf1 = w1.reshape(K, K, LANE, LANE)[:, :, :C2, :].reshape(K, K * C2, LANE)
    wf1 = jnp.pad(wf1, ((0, 0), (0, LANE - K * C2), (0, 0))).astype(bf16)

    def _pairs(w5):
        return jnp.stack([jnp.concatenate([w5[0], w5[1]], axis=0),
                          jnp.concatenate([w5[2], w5[3]], axis=0)])
    wap, wa4 = _pairs(wa), wa[4]
    wbp, wb4 = _pairs(wb), wb[4]
    wf1p, wf14 = _pairs(wf1), wf1[4]

    xv = jnp.pad(
        jnp.transpose(x.astype(bf16), (2, 0, 1, 3)).reshape(H1, n, C0 * W1C),
        ((0, 0), (0, 0), (0, LANE - C0 * W1C)))

    out = pl.pallas_call(
        _lenet_kernel,
        out_shape=jax.ShapeDtypeStruct((n, 10), jnp.float32),
        grid=(n // BIMG,),
        in_specs=[
            pl.BlockSpec((H1, BIMG, LANE), lambda i: (0, i, 0)),
            pl.BlockSpec((2, 2 * LANE, 2 * LANE), lambda i: (0, 0, 0)),
            pl.BlockSpec((LANE, 2 * LANE), lambda i: (0, 0)),
            pl.BlockSpec((1, 2 * LANE), lambda i: (0, 0)),
            pl.BlockSpec((2 * LANE, LANE), lambda i: (0, 0)),
            pl.BlockSpec((2 * LANE, LANE), lambda i: (0, 0)),
            pl.BlockSpec((2, 2 * LANE, 2 * LANE), lambda i: (0, 0, 0)),
            pl.BlockSpec((LANE, 2 * LANE), lambda i: (0, 0)),
            pl.BlockSpec((1, 2 * LANE), lambda i: (0, 0)),
            pl.BlockSpec((2 * LANE, LANE), lambda i: (0, 0)),
            pl.BlockSpec((2 * LANE, LANE), lambda i: (0, 0)),
            pl.BlockSpec((2, 2 * LANE, LANE), lambda i: (0, 0, 0)),
            pl.BlockSpec((LANE, LANE), lambda i: (0, 0)),
            pl.BlockSpec((1, LANE), lambda i: (0, 0)),
            pl.BlockSpec((LANE, LANE), lambda i: (0, 0)),
            pl.BlockSpec((1, LANE), lambda i: (0, 0)),
            pl.BlockSpec((LANE, LANE), lambda i: (0, 0)),
            pl.BlockSpec((1, LANE), lambda i: (0, 0)),
        ],
        out_specs=pl.BlockSpec((BIMG, 10), lambda i: (i, 0)),
        compiler_params=pltpu.CompilerParams(
            dimension_semantics=("parallel",),
            allow_input_fusion=(True,) + (False,) * 17,
            vmem_limit_bytes=60 * 1024 * 1024,
        ),
    )(xv, wap, wa4, ba,
      jnp.asarray(_S1E, bf16), jnp.asarray(_S1O, bf16),
      wbp, wb4, bb,
      jnp.asarray(_S2E, bf16), jnp.asarray(_S2O, bf16),
      wf1p, wf14, b1, w2.astype(bf16), b2, w3.astype(bf16), b3)
    return out
